# baseline (speedup 1.0000x reference)
import functools

import jax
import jax.numpy as jnp
from jax import lax
from jax.experimental import pallas as pl
from jax.experimental.pallas import tpu as pltpu
from jax.experimental.pallas import tpu_sc as plsc

_OFFSET = 2


def kernel(input_ids, pos_table):
    bsz, seq_len = input_ids.shape
    dim = pos_table.shape[-1]

    info = plsc.get_sparse_core_info()
    num_cores, num_subcores = info.num_cores, info.num_subcores
    num_lanes = info.num_lanes
    num_workers = num_cores * num_subcores
    rows_per_worker = seq_len // num_workers
    chunk_rows = 32
    n_chunks = rows_per_worker // chunk_rows


    @functools.partial(
        pl.kernel,
        mesh=plsc.VectorSubcoreMesh(core_axis_name="c", subcore_axis_name="s"),
        out_type=jax.ShapeDtypeStruct((bsz, seq_len, dim), jnp.float32),
        scratch_types=[
            pltpu.VMEM((chunk_rows, dim), jnp.float32),
            pltpu.VMEM((chunk_rows, dim), jnp.float32),
            pltpu.VMEM((chunk_rows, dim), jnp.float32),
            pltpu.VMEM((8, dim), jnp.float32),
            pltpu.VMEM_SHARED((num_subcores, 2, 8, dim), jnp.float32),
            pltpu.SemaphoreType.DMA,
            pltpu.SemaphoreType.DMA,
            pltpu.SemaphoreType.DMA,
            pltpu.SemaphoreType.DMA,
            pltpu.SemaphoreType.DMA,
        ],
    )
    def pe_kernel(
        table_hbm,
        out_hbm,
        buf0,
        buf1,
        buf2,
        side,
        shared,
        in_sem,
        side_sem,
        out_sem,
        sp_in_sem,
        sp_out_sem,
    ):
        wid = lax.axis_index("s") * num_cores + lax.axis_index("c")
        base = wid * rows_per_worker
        bufs = (buf0, buf1, buf2)

        def read(g, buf):
            row0 = base + g * chunk_rows
            return pltpu.async_copy(
                table_hbm.at[pl.ds(row0, chunk_rows), :], buf, in_sem
            )

        n_j = dim // (num_lanes * 8)

        def shift_rows(dst, dst_r0, src, src_r0, n_rows):
            def col_body(j, _):
                c0 = j * (num_lanes * 8)
                for jj in range(8):
                    c = c0 + jj * num_lanes
                    for i in range(n_rows):
                        dst[dst_r0 + i, pl.ds(c, num_lanes)] = src[
                            src_r0 + i, pl.ds(c, num_lanes)
                        ]
                return 0

            lax.fori_loop(0, n_j, col_body, 0)

        def write(g, buf, r0, n_rows, n_batches):
            row0 = base + g * chunk_rows
            return [
                pltpu.async_copy(
                    buf.at[pl.ds(r0, n_rows), :],
                    out_hbm.at[b, pl.ds(row0 + r0, n_rows), :],
                    out_sem,
                )
                for b in range(n_batches)
            ]

        sid = lax.axis_index("s")
        n_spmem = max(n_chunks - 2, 0)
        sub_rows = 8
        subs = chunk_rows // sub_rows

        def sub_slice(u):
            return pl.ds(base + u * sub_rows, sub_rows)

        rds = {0: read(0, bufs[0])}
        if n_chunks > 1:
            rds[1] = read(1, bufs[1])
        side_rd = pltpu.async_copy(
            table_hbm.at[pl.ds(base + rows_per_worker, 8), :], side, side_sem
        )
        pending = {}
        hop = {}
        cp2 = {}
        for g in range(n_chunks):
            buf = bufs[g % 3]
            n_batches = bsz - 1 if g < n_spmem else bsz
            if g == 0:
                rds[0].wait()
            for k in range(chunk_rows // 8 - 1):
                shift_rows(buf, k * 8, buf, k * 8 + _OFFSET, 8)
            bulk = write(g, buf, 0, chunk_rows - 8, n_batches)
            shift_rows(
                buf, chunk_rows - 8, buf, chunk_rows - 8 + _OFFSET, 8 - _OFFSET
            )
            if g + 1 < n_chunks:
                rds[g + 1].wait()
                nxt = bufs[(g + 1) % 3]
            else:
                side_rd.wait()
                nxt = side
            shift_rows(buf, chunk_rows - _OFFSET, nxt, 0, _OFFSET)
            pending[g] = bulk + write(g, buf, chunk_rows - 8, 8, n_batches)
            if g - 1 in pending:
                for c in pending.pop(g - 1):
                    c.wait()
            h = g - 1
            if 0 <= h < n_spmem:
                for u in range(subs * h, subs * (h + 1)):
                    if u - 2 in cp2:
                        cp2.pop(u - 2).wait()
                    hop[u] = pltpu.async_copy(
                        out_hbm.at[0, sub_slice(u), :],
                        shared.at[sid, u % 2],
                        sp_in_sem,
                    )
                    if u - 1 in hop:
                        hop.pop(u - 1).wait()
                        cp2[u - 1] = pltpu.async_copy(
                            shared.at[sid, (u - 1) % 2],
                            out_hbm.at[bsz - 1, sub_slice(u - 1), :],
                            sp_out_sem,
                        )
            if g + 2 < n_chunks:
                rds[g + 2] = read(g + 2, bufs[(g + 2) % 3])
        for copies in pending.values():
            for c in copies:
                c.wait()
        for u in sorted(hop):
            hop.pop(u).wait()
            cp2[u] = pltpu.async_copy(
                shared.at[sid, u % 2],
                out_hbm.at[bsz - 1, sub_slice(u), :],
                sp_out_sem,
            )
        for u in sorted(cp2):
            cp2.pop(u).wait()

    return pe_kernel(pos_table)


# speedup vs baseline: 1.1428x; 1.1428x over previous
import functools

import jax
import jax.numpy as jnp
from jax import lax
from jax.experimental import pallas as pl
from jax.experimental.pallas import tpu as pltpu
from jax.experimental.pallas import tpu_sc as plsc

_OFFSET = 2


def kernel(input_ids, pos_table):
    bsz, seq_len = input_ids.shape
    dim = pos_table.shape[-1]

    info = plsc.get_sparse_core_info()
    num_cores, num_subcores = info.num_cores, info.num_subcores
    num_lanes = info.num_lanes
    num_workers = num_cores * num_subcores
    rows_per_worker = seq_len // num_workers
    chunk_rows = 32
    n_chunks = rows_per_worker // chunk_rows


    @functools.partial(
        pl.kernel,
        mesh=plsc.VectorSubcoreMesh(core_axis_name="c", subcore_axis_name="s"),
        out_type=jax.ShapeDtypeStruct((bsz, seq_len, dim), jnp.float32),
        scratch_types=[
            pltpu.VMEM((chunk_rows, dim), jnp.float32),
            pltpu.VMEM((chunk_rows, dim), jnp.float32),
            pltpu.VMEM((chunk_rows, dim), jnp.float32),
            pltpu.VMEM((8, dim), jnp.float32),
            pltpu.SemaphoreType.DMA,
            pltpu.SemaphoreType.DMA,
            pltpu.SemaphoreType.DMA,
        ],
    )
    def pe_kernel(
        table_hbm, out_hbm, buf0, buf1, buf2, side, in_sem, side_sem, out_sem
    ):
        wid = lax.axis_index("s") * num_cores + lax.axis_index("c")
        base = wid * rows_per_worker
        bufs = (buf0, buf1, buf2)

        def read(g, buf):
            row0 = base + g * chunk_rows
            return pltpu.async_copy(
                table_hbm.at[pl.ds(row0, chunk_rows), :], buf, in_sem
            )

        n_j = dim // (num_lanes * 8)

        def shift_rows(dst, dst_r0, src, src_r0, n_rows):
            def col_body(j, _):
                c0 = j * (num_lanes * 8)
                for jj in range(8):
                    c = c0 + jj * num_lanes
                    for i in range(n_rows):
                        dst[dst_r0 + i, pl.ds(c, num_lanes)] = src[
                            src_r0 + i, pl.ds(c, num_lanes)
                        ]
                return 0

            lax.fori_loop(0, n_j, col_body, 0)

        def write(g, buf, r0, n_rows):
            row0 = base + g * chunk_rows
            return [
                pltpu.async_copy(
                    buf.at[pl.ds(r0, n_rows), :],
                    out_hbm.at[b, pl.ds(row0 + r0, n_rows), :],
                    out_sem,
                )
                for b in range(bsz)
            ]

        rds = {0: read(0, bufs[0])}
        if n_chunks > 1:
            rds[1] = read(1, bufs[1])
        side_rd = pltpu.async_copy(
            table_hbm.at[pl.ds(base + rows_per_worker, 8), :], side, side_sem
        )
        pending = {}
        for g in range(n_chunks):
            buf = bufs[g % 3]
            if g == 0:
                rds[0].wait()
            for k in range(chunk_rows // 8 - 1):
                shift_rows(buf, k * 8, buf, k * 8 + _OFFSET, 8)
            bulk = write(g, buf, 0, chunk_rows - 8)
            shift_rows(
                buf, chunk_rows - 8, buf, chunk_rows - 8 + _OFFSET, 8 - _OFFSET
            )
            if g + 1 < n_chunks:
                rds[g + 1].wait()
                nxt = bufs[(g + 1) % 3]
            else:
                side_rd.wait()
                nxt = side
            shift_rows(buf, chunk_rows - _OFFSET, nxt, 0, _OFFSET)
            pending[g] = bulk + write(g, buf, chunk_rows - 8, 8)
            if g - 1 in pending:
                for c in pending.pop(g - 1):
                    c.wait()
            if g + 2 < n_chunks:
                rds[g + 2] = read(g + 2, bufs[(g + 2) % 3])
        for copies in pending.values():
            for c in copies:
                c.wait()

    return pe_kernel(pos_table)
